# Initial kernel scaffold; baseline (speedup 1.0000x reference)
import jax
import jax.numpy as jnp
from jax.experimental import pallas as pl


def kernel(feat, edge_index, weight, bias):
    raise NotImplementedError("write your pallas kernel here")


# speedup vs baseline: 6.4916x; 6.4916x over previous
import functools

import jax
import jax.numpy as jnp
from jax import lax
from jax.experimental import pallas as pl
from jax.experimental.pallas import tpu as pltpu
from jax.experimental.pallas import tpu_sc as plsc

_NC = 2
_NS = 16
_NW = _NC * _NS
_B = 128


def _zero_vmem_1d(ref, n):
    z = jnp.zeros((16,), jnp.float32)
    for k in range(n // 16):
        ref[pl.ds(k * 16, 16)] = z


@functools.lru_cache(maxsize=None)
def _build_deg(n, e, npad):
    nb = e // _B
    rpt = npad // _NS

    mesh = plsc.VectorSubcoreMesh(core_axis_name="c", subcore_axis_name="s")

    @functools.partial(
        pl.kernel,
        out_type=jax.ShapeDtypeStruct((_NC * npad,), jnp.float32),
        mesh=mesh,
        scratch_types=[
            pltpu.VMEM_SHARED((npad,), jnp.float32),
            pltpu.VMEM((2, _B), jnp.int32),
            pltpu.VMEM((_B,), jnp.float32),
            pltpu.VMEM((rpt,), jnp.float32),
        ],
    )
    def deg_kernel(ei_hbm, deg_hbm, deg_sh, idx_v, ones_v, zer_v):
        c = lax.axis_index("c")
        s = lax.axis_index("s")
        wid = s * _NC + c
        nblk = (nb - 1 - wid) // _NW + 1

        _zero_vmem_1d(zer_v, rpt)
        one = jnp.ones((16,), jnp.float32)
        for k in range(_B // 16):
            ones_v[pl.ds(k * 16, 16)] = one
        pltpu.sync_copy(zer_v, deg_sh.at[pl.ds(s * rpt, rpt)])
        plsc.subcore_barrier()

        def blk(i, carry):
            off = (wid + i * _NW) * _B
            pltpu.sync_copy(ei_hbm.at[:, pl.ds(off, _B)], idx_v)
            pltpu.sync_copy(ones_v, deg_sh.at[idx_v.at[0]], add=True)
            return carry

        lax.fori_loop(0, nblk, blk, 0)

        plsc.subcore_barrier()
        pltpu.sync_copy(deg_sh.at[pl.ds(s * rpt, rpt)],
                        deg_hbm.at[pl.ds(c * npad + s * rpt, rpt)])

    return deg_kernel


@functools.lru_cache(maxsize=None)
def _build_agg(n, d, e, npadr):
    nb = e // _B
    npr = npadr // _NS
    zr = 160

    mesh = plsc.VectorSubcoreMesh(core_axis_name="c", subcore_axis_name="s")

    @functools.partial(
        pl.kernel,
        out_type=jax.ShapeDtypeStruct((_NC, npadr, d), jnp.float32),
        mesh=mesh,
        scratch_types=[
            pltpu.VMEM_SHARED((npadr, d), jnp.float32),
            pltpu.VMEM((2, _B), jnp.int32),
            pltpu.VMEM((_B, d), jnp.float32),
            pltpu.VMEM((zr, d), jnp.float32),
            pltpu.SemaphoreType.DMA,
        ],
    )
    def agg_kernel(h_hbm, ei_hbm, agg_hbm, agg_sh, idx2, rows, zrow, sem):
        c = lax.axis_index("c")
        s = lax.axis_index("s")
        wid = s * _NC + c
        nblk = (nb - 1 - wid) // _NW + 1
        rbase = s * npr

        for r in range(zr):
            _zero_vmem_1d(zrow.at[r], d)
        k = 0
        while k < npr:
            sz = min(zr, npr - k)
            pltpu.sync_copy(zrow.at[pl.ds(0, sz)],
                            agg_sh.at[pl.ds(rbase + k, sz)])
            k += sz
        plsc.subcore_barrier()

        def blk(i, carry):
            off = (wid + i * _NW) * _B
            pltpu.sync_copy(ei_hbm.at[:, pl.ds(off, _B)], idx2)
            pltpu.async_copy(h_hbm.at[idx2.at[0]], rows, sem).wait()
            pltpu.sync_copy(rows, agg_sh.at[idx2.at[1]], add=True)
            return carry

        lax.fori_loop(0, nblk, blk, 0)

        plsc.subcore_barrier()
        pltpu.sync_copy(agg_sh.at[pl.ds(rbase, npr)],
                        agg_hbm.at[c, pl.ds(rbase, npr)])

    return agg_kernel


def _mm_body(feat_ref, w_ref, deg_ref, h_ref):
    deg = deg_ref[0] + deg_ref[1]
    norm = lax.rsqrt(jnp.maximum(deg, 1.0))
    h = lax.dot_general(feat_ref[...], w_ref[...], (((1,), (0,)), ((), ())),
                        precision=lax.Precision.HIGHEST,
                        preferred_element_type=jnp.float32)
    h_ref[...] = h * norm


def _fin_body(aggp_ref, deg_ref, b_ref, out_ref):
    agg = aggp_ref[0] + aggp_ref[1]
    norm = lax.rsqrt(jnp.maximum(deg_ref[0] + deg_ref[1], 1.0))
    out_ref[...] = agg * norm + b_ref[...]


def kernel(feat, edge_index, weight, bias):
    n, d = feat.shape
    e = edge_index.shape[1]
    npad = ((n + _NS * 16 - 1) // (_NS * 16)) * (_NS * 16)
    npadr = _NS * (((n + _NS * 8 - 1) // (_NS * 8)) * 8)

    degf = _build_deg(n, e, npad)(edge_index)
    degs = degf.reshape(_NC, npad)[:, :n, None]

    h = pl.pallas_call(
        _mm_body,
        out_shape=jax.ShapeDtypeStruct((n, d), jnp.float32),
    )(feat, weight, degs)

    aggp = _build_agg(n, d, e, npadr)(h, edge_index)

    out = pl.pallas_call(
        _fin_body,
        out_shape=jax.ShapeDtypeStruct((n, d), jnp.float32),
    )(aggp[:, :n], degs, bias.reshape(1, d))
    return out
